# Initial kernel scaffold; baseline (speedup 1.0000x reference)
import jax
import jax.numpy as jnp
from jax.experimental import pallas as pl


def kernel(x, edge_index, batch, W_self, W_neigh, b):
    raise NotImplementedError("write your pallas kernel here")


# speedup vs baseline: 4.6098x; 4.6098x over previous
import functools

import jax
import jax.numpy as jnp
from jax import lax
from jax.experimental import pallas as pl
from jax.experimental.pallas import tpu as pltpu
from jax.experimental.pallas import tpu_sc as plsc

N_NODES = 10000
N_EDGES = 320000
D_FEAT = 128

NC = 2
NS = 16
NW = NC * NS
CHUNK = 128
CHUNKS = 80
E_PAD = NW * CHUNKS * CHUNK
ROWS_PER_TILE = 632
N_ACC = ROWS_PER_TILE * NS
JUNK = N_NODES
L = 16


def _sc_accumulate(x, sd3, zeros_init):
    mesh = plsc.VectorSubcoreMesh(core_axis_name="c", subcore_axis_name="s")

    @functools.partial(
        pl.kernel,
        out_type=(
            jax.ShapeDtypeStruct((NC, N_ACC, D_FEAT), jnp.float32),
            jax.ShapeDtypeStruct((NC, NS, N_ACC), jnp.float32),
        ),
        mesh=mesh,
        scratch_types=[
            pltpu.VMEM((2, 4, CHUNK), jnp.int32),
            pltpu.VMEM((CHUNK, D_FEAT), jnp.float32),
            pltpu.VMEM((CHUNK, D_FEAT), jnp.float32),
            pltpu.VMEM((N_ACC,), jnp.float32),
            pltpu.VMEM_SHARED((N_ACC, D_FEAT), jnp.float32),
            pltpu.SemaphoreType.DMA,
            pltpu.SemaphoreType.DMA,
            pltpu.SemaphoreType.DMA,
        ],
        compiler_params=pltpu.CompilerParams(needs_layout_passes=False),
    )
    def sc_kernel(x_hbm, sd_hbm, zero_hbm, acc_hbm, deg_hbm,
                  sd_w, rows_a, rows_b, hist_v, acc_sh, sem_a, sem_b, sem_i):
        c = lax.axis_index("c")
        s = lax.axis_index("s")
        wid = s * NC + c

        pltpu.sync_copy(zero_hbm,
                        acc_sh.at[pl.ds(s * ROWS_PER_TILE, ROWS_PER_TILE)])

        zeros16 = jnp.zeros((L,), jnp.float32)

        def zero_body(i, _):
            hist_v[pl.ds(i * L, L)] = zeros16
            return 0

        lax.fori_loop(0, N_ACC // L, zero_body, 0)
        plsc.subcore_barrier()

        ones16 = jnp.ones((L,), jnp.float32)

        def hist_rows(slot, r):
            for m in range(CHUNK // L):
                d = sd_w[slot, r, pl.ds(m * L, L)]
                plsc.addupdate_scatter(hist_v, [d], ones16)

        pltpu.async_copy(sd_hbm.at[wid].at[pl.ds(0, 4)], sd_w.at[0], sem_i)

        def body(j, _):
            slot = lax.rem(j, 2)
            nslot = lax.rem(j + 1, 2)
            pltpu.make_async_copy(sd_hbm.at[wid].at[pl.ds(4 * j, 4)],
                                  sd_w.at[slot], sem_i).wait()
            pltpu.async_copy(sd_hbm.at[wid].at[pl.ds(4 * j + 4, 4)],
                             sd_w.at[nslot], sem_i)
            pltpu.async_copy(x_hbm.at[sd_w.at[slot, 0]], rows_a, sem_a)
            pltpu.async_copy(x_hbm.at[sd_w.at[slot, 2]], rows_b, sem_b)
            pltpu.make_async_copy(x_hbm.at[sd_w.at[slot, 0]], rows_a,
                                  sem_a).wait()
            pltpu.sync_copy(rows_a, acc_sh.at[sd_w.at[slot, 1]], add=True)
            hist_rows(slot, 1)
            pltpu.make_async_copy(x_hbm.at[sd_w.at[slot, 2]], rows_b,
                                  sem_b).wait()
            pltpu.sync_copy(rows_b, acc_sh.at[sd_w.at[slot, 3]], add=True)
            hist_rows(slot, 3)
            return 0

        lax.fori_loop(0, CHUNKS // 2, body, 0)
        pltpu.make_async_copy(sd_hbm.at[wid].at[pl.ds(4 * CHUNKS // 2, 4)],
                              sd_w.at[0], sem_i).wait()
        plsc.subcore_barrier()

        pltpu.sync_copy(acc_sh.at[pl.ds(s * ROWS_PER_TILE, ROWS_PER_TILE)],
                        acc_hbm.at[c].at[pl.ds(s * ROWS_PER_TILE,
                                               ROWS_PER_TILE)])
        pltpu.sync_copy(hist_v, deg_hbm.at[c].at[s])

    return sc_kernel(x, sd3, zeros_init)


def _tc_body(x_ref, acc_ref, deg_ref, ws_ref, wn_ref, b_ref, o_ref):
    feats = acc_ref[0] + acc_ref[1]
    deg = jnp.maximum(deg_ref[...], 1.0)
    agg = jnp.dot(feats / deg, wn_ref[...], preferred_element_type=jnp.float32)
    z = jnp.dot(x_ref[...], ws_ref[...], preferred_element_type=jnp.float32)
    o_ref[...] = jnp.maximum(z + agg + b_ref[...], 0.0)


def _tc_finish(x, acc, deg, W_self, W_neigh, b):
    R = 2000
    grid = N_NODES // R
    return pl.pallas_call(
        _tc_body,
        grid=(grid,),
        in_specs=[
            pl.BlockSpec((R, D_FEAT), lambda i: (i, 0)),
            pl.BlockSpec((NC, R, D_FEAT), lambda i: (0, i, 0)),
            pl.BlockSpec((R, 1), lambda i: (i, 0)),
            pl.BlockSpec((D_FEAT, D_FEAT), lambda i: (0, 0)),
            pl.BlockSpec((D_FEAT, D_FEAT), lambda i: (0, 0)),
            pl.BlockSpec((1, D_FEAT), lambda i: (0, 0)),
        ],
        out_specs=pl.BlockSpec((R, D_FEAT), lambda i: (i, 0)),
        out_shape=jax.ShapeDtypeStruct((N_NODES, D_FEAT), jnp.float32),
    )(x, acc, deg, W_self, W_neigh, b.reshape(1, D_FEAT))


def kernel(x, edge_index, batch, W_self, W_neigh, b):
    src = edge_index[0].astype(jnp.int32)
    dst = edge_index[1].astype(jnp.int32)

    pad = E_PAD - N_EDGES
    src_p = jnp.concatenate([src, jnp.zeros((pad,), jnp.int32)])
    dst_p = jnp.concatenate([dst, jnp.full((pad,), JUNK, jnp.int32)])
    src3 = src_p.reshape(NW, CHUNKS, CHUNK)
    dst3 = dst_p.reshape(NW, CHUNKS, CHUNK)
    src3 = jnp.concatenate(
        [src3, jnp.zeros((NW, 2, CHUNK), jnp.int32)], axis=1)
    dst3 = jnp.concatenate(
        [dst3, jnp.full((NW, 2, CHUNK), JUNK, jnp.int32)], axis=1)
    sd3 = jnp.stack([src3, dst3], axis=2).reshape(NW, 2 * (CHUNKS + 2), CHUNK)

    zeros_init = jnp.zeros((ROWS_PER_TILE, D_FEAT), jnp.float32)

    acc, deg_parts = _sc_accumulate(x, sd3, zeros_init)
    deg = deg_parts.sum(axis=(0, 1))[:N_NODES, None]
    node_emb = _tc_finish(x, acc, deg, W_self, W_neigh, b)
    return node_emb, batch
